# Initial kernel scaffold; baseline (speedup 1.0000x reference)
import jax
import jax.numpy as jnp
from jax.experimental import pallas as pl


def kernel(pos, edge_index, nbr_shift):
    raise NotImplementedError("write your pallas kernel here")


# speedup vs baseline: 3.7204x; 3.7204x over previous
import functools

import jax
import jax.numpy as jnp
from jax import lax
from jax.experimental import pallas as pl
from jax.experimental.pallas import tpu as pltpu
from jax.experimental.pallas import tpu_sc as plsc

_NUM_BASIS = 8
_R_MAX = 6.0
_NC = 2
_NS = 16
_NW = _NC * _NS
_CHUNK = 2000

_HALF_PI = 1.5707963267948966
_PREF = 2.0 / _R_MAX


def _rsqrt(s):
    si = plsc.bitcast(s, jnp.int32)
    yi = jnp.int32(0x5F3759DF) - lax.shift_right_logical(si, 1)
    y = plsc.bitcast(yi, jnp.float32)
    for _ in range(3):
        y = y * (1.5 - 0.5 * s * y * y)
    return y


def _basis_block(xi, yi, zi, xj, yj, zj, sx, sy, sz):
    dx = xi + sx - xj
    dy = yi + sy - yj
    dz = zi + sz - zj
    s = dx * dx + dy * dy + dz * dz
    invx = _rsqrt(s)
    x = s * invx
    inside = s < (_R_MAX * _R_MAX)
    u = jnp.minimum(x * (1.0 / _R_MAX), 1.0)
    t = u * _HALF_PI
    t2 = t * t
    sh = t * (1.0 + t2 * (-1.0 / 6.0 + t2 * (1.0 / 120.0
         + t2 * (-1.0 / 5040.0 + t2 * (1.0 / 362880.0)))))
    ch = 1.0 + t2 * (-0.5 + t2 * (1.0 / 24.0 + t2 * (-1.0 / 720.0
         + t2 * (1.0 / 40320.0 + t2 * (-1.0 / 3628800.0)))))
    s1 = 2.0 * sh * ch
    c1 = 1.0 - 2.0 * sh * sh
    tc = 2.0 * c1
    u2 = u * u
    u6 = u2 * u2 * u2
    cut = 1.0 + u6 * (-28.0 + u * (48.0 - 21.0 * u))
    f = jnp.where(inside, cut * invx * _PREF, 0.0)
    return f, tc, s1


def _make_sc_kernel(n_nodes, n_edges):
    per_w = n_edges // _NW
    n_chunks = per_w // _CHUNK
    c = _CHUNK
    mesh = plsc.VectorSubcoreMesh(core_axis_name="c", subcore_axis_name="s")

    @functools.partial(
        pl.kernel,
        out_type=jax.ShapeDtypeStruct((n_edges, _NUM_BASIS), jnp.float32),
        mesh=mesh,
        scratch_types=[
            pltpu.VMEM((c,), jnp.int32),
            pltpu.VMEM((c,), jnp.int32),
            pltpu.VMEM((c, 8), jnp.float32),
            pltpu.VMEM((c, 8), jnp.float32),
            pltpu.VMEM((c, 3), jnp.float32),
            pltpu.VMEM((c, _NUM_BASIS), jnp.float32),
            pltpu.SemaphoreType.DMA,
            pltpu.SemaphoreType.DMA,
        ],
        compiler_params=pltpu.CompilerParams(needs_layout_passes=False, use_tc_tiling_on_sc=False),
    )
    def sc_kernel(pos_hbm, ej_hbm, ei_hbm, shift_hbm, out_hbm,
                  idxj_v, idxi_v, pj_v, pi_v, sh_v, o_v, sem_j, sem_i):
        wid = lax.axis_index("s") * _NC + lax.axis_index("c")
        lanes = lax.iota(jnp.int32, 16)
        zeros16 = jnp.zeros((16,), jnp.int32)

        def chunk_body(k, _):
            base = wid * per_w + k * c
            pltpu.sync_copy(ej_hbm.at[pl.ds(base, c)], idxj_v)
            pltpu.sync_copy(ei_hbm.at[pl.ds(base, c)], idxi_v)
            cj = pltpu.async_copy(pos_hbm.at[idxj_v], pj_v, sem_j)
            ci = pltpu.async_copy(pos_hbm.at[idxi_v], pi_v, sem_i)
            pltpu.sync_copy(shift_hbm.at[pl.ds(base, c), :], sh_v)
            cj.wait()
            ci.wait()

            def group_body(g, _):
                rows = g * 16 + lanes
                xi = plsc.load_gather(pi_v, [rows, zeros16])
                yi = plsc.load_gather(pi_v, [rows, zeros16 + 1])
                zi = plsc.load_gather(pi_v, [rows, zeros16 + 2])
                xj = plsc.load_gather(pj_v, [rows, zeros16])
                yj = plsc.load_gather(pj_v, [rows, zeros16 + 1])
                zj = plsc.load_gather(pj_v, [rows, zeros16 + 2])
                sx = plsc.load_gather(sh_v, [rows, zeros16])
                sy = plsc.load_gather(sh_v, [rows, zeros16 + 1])
                sz = plsc.load_gather(sh_v, [rows, zeros16 + 2])
                f, tc, s1 = _basis_block(xi, yi, zi, xj, yj, zj, sx, sy, sz)
                sm = s1
                smm = jnp.zeros((16,), jnp.float32)
                for n in range(_NUM_BASIS):
                    plsc.store_scatter(o_v, [rows, zeros16 + n], sm * f)
                    sm, smm = tc * sm - smm, sm
                return 0

            lax.fori_loop(0, c // 16, group_body, 0)
            pltpu.sync_copy(o_v, out_hbm.at[pl.ds(base, c), :])
            return 0

        lax.fori_loop(0, n_chunks, chunk_body, 0)

    return sc_kernel


def kernel(pos, edge_index, nbr_shift):
    n_nodes = pos.shape[0]
    n_edges = edge_index.shape[1]
    pos4 = jnp.pad(pos, ((0, 0), (0, 5)))
    ej = edge_index[0]
    ei = edge_index[1]
    sc = _make_sc_kernel(n_nodes, n_edges)
    return sc(pos4, ej, ei, nbr_shift)


# speedup vs baseline: 4.2605x; 1.1451x over previous
import functools

import jax
import jax.numpy as jnp
from jax import lax
from jax.experimental import pallas as pl
from jax.experimental.pallas import tpu as pltpu
from jax.experimental.pallas import tpu_sc as plsc

_NUM_BASIS = 8
_R_MAX = 6.0
_NC = 2
_NS = 16
_NW = _NC * _NS
_CHUNK = 2048

_HALF_PI = 1.5707963267948966
_PREF = 2.0 / _R_MAX


def _rsqrt(s):
    si = plsc.bitcast(s, jnp.int32)
    yi = jnp.int32(0x5F3759DF) - lax.shift_right_logical(si, 1)
    y = plsc.bitcast(yi, jnp.float32)
    for _ in range(3):
        y = y * (1.5 - 0.5 * s * y * y)
    return y


def _basis_block(xi, yi, zi, xj, yj, zj, sx, sy, sz):
    dx = xi + sx - xj
    dy = yi + sy - yj
    dz = zi + sz - zj
    s = dx * dx + dy * dy + dz * dz
    invx = _rsqrt(s)
    x = s * invx
    inside = s < (_R_MAX * _R_MAX)
    u = jnp.minimum(x * (1.0 / _R_MAX), 1.0)
    t = u * _HALF_PI
    t2 = t * t
    sh = t * (1.0 + t2 * (-1.0 / 6.0 + t2 * (1.0 / 120.0
         + t2 * (-1.0 / 5040.0 + t2 * (1.0 / 362880.0)))))
    ch = 1.0 + t2 * (-0.5 + t2 * (1.0 / 24.0 + t2 * (-1.0 / 720.0
         + t2 * (1.0 / 40320.0 + t2 * (-1.0 / 3628800.0)))))
    s1 = 2.0 * sh * ch
    c1 = 1.0 - 2.0 * sh * sh
    tc = 2.0 * c1
    u2 = u * u
    u6 = u2 * u2 * u2
    cut = 1.0 + u6 * (-28.0 + u * (48.0 - 21.0 * u))
    f = jnp.where(inside, cut * invx * _PREF, 0.0)
    return f, tc, s1


def _make_sc_kernel(n_edges):
    c = _CHUNK
    n_chunks = n_edges // c
    k_max = -(-n_chunks // _NW)
    shift_rows = (c * 3) // 128
    out_rows = (c * _NUM_BASIS) // 128
    mesh = plsc.VectorSubcoreMesh(core_axis_name="c", subcore_axis_name="s")

    @functools.partial(
        pl.kernel,
        out_type=jax.ShapeDtypeStruct((n_edges * _NUM_BASIS // 128, 128),
                                      jnp.float32),
        mesh=mesh,
        scratch_types=[
            pltpu.VMEM((c,), jnp.int32),
            pltpu.VMEM((c,), jnp.int32),
            pltpu.VMEM((c, 8), jnp.float32),
            pltpu.VMEM((c, 8), jnp.float32),
            pltpu.VMEM((shift_rows, 128), jnp.float32),
            pltpu.VMEM((out_rows, 128), jnp.float32),
            pltpu.SemaphoreType.DMA,
            pltpu.SemaphoreType.DMA,
        ],
        compiler_params=pltpu.CompilerParams(needs_layout_passes=False,
                                             use_tc_tiling_on_sc=False),
    )
    def sc_kernel(pos_hbm, ej_hbm, ei_hbm, shift_hbm, out_hbm,
                  idxj_v, idxi_v, pj_v, pi_v, sh_v, o_v, sem_j, sem_i):
        wid = lax.axis_index("s") * _NC + lax.axis_index("c")
        lanes = lax.iota(jnp.int32, 16)
        zeros16 = jnp.zeros((16,), jnp.int32)

        def chunk_body(k, _):
            ck = wid + k * _NW

            @pl.when(ck < n_chunks)
            def _():
                base = ck * c
                pltpu.sync_copy(ej_hbm.at[pl.ds(base, c)], idxj_v)
                pltpu.sync_copy(ei_hbm.at[pl.ds(base, c)], idxi_v)
                cj = pltpu.async_copy(pos_hbm.at[idxj_v], pj_v, sem_j)
                ci = pltpu.async_copy(pos_hbm.at[idxi_v], pi_v, sem_i)
                pltpu.sync_copy(
                    shift_hbm.at[pl.ds(ck * shift_rows, shift_rows), :], sh_v)
                cj.wait()
                ci.wait()

                def group_body(g, _):
                    rows = g * 16 + lanes
                    xi = plsc.load_gather(pi_v, [rows, zeros16])
                    yi = plsc.load_gather(pi_v, [rows, zeros16 + 1])
                    zi = plsc.load_gather(pi_v, [rows, zeros16 + 2])
                    xj = plsc.load_gather(pj_v, [rows, zeros16])
                    yj = plsc.load_gather(pj_v, [rows, zeros16 + 1])
                    zj = plsc.load_gather(pj_v, [rows, zeros16 + 2])
                    r3 = rows * 3
                    sx = plsc.load_gather(
                        sh_v, [lax.shift_right_logical(r3, 7), r3 & 127])
                    r3 = r3 + 1
                    sy = plsc.load_gather(
                        sh_v, [lax.shift_right_logical(r3, 7), r3 & 127])
                    r3 = r3 + 1
                    sz = plsc.load_gather(
                        sh_v, [lax.shift_right_logical(r3, 7), r3 & 127])
                    f, tc, s1 = _basis_block(xi, yi, zi, xj, yj, zj,
                                             sx, sy, sz)
                    sm = s1
                    smm = jnp.zeros((16,), jnp.float32)
                    r8 = rows * _NUM_BASIS
                    for n in range(_NUM_BASIS):
                        rn = r8 + n
                        plsc.store_scatter(
                            o_v, [lax.shift_right_logical(rn, 7), rn & 127],
                            sm * f)
                        sm, smm = tc * sm - smm, sm
                    return 0

                lax.fori_loop(0, c // 16, group_body, 0)
                pltpu.sync_copy(
                    o_v, out_hbm.at[pl.ds(ck * out_rows, out_rows), :])

            return 0

        lax.fori_loop(0, k_max, chunk_body, 0)

    return sc_kernel


def kernel(pos, edge_index, nbr_shift):
    n_edges = edge_index.shape[1]
    pos8 = jnp.pad(pos, ((0, 0), (0, 5)))
    ej = edge_index[0]
    ei = edge_index[1]
    shift128 = nbr_shift.reshape(n_edges * 3 // 128, 128)
    sc = _make_sc_kernel(n_edges)
    out128 = sc(pos8, ej, ei, shift128)
    return out128.reshape(n_edges, _NUM_BASIS)
